# Initial kernel scaffold; baseline (speedup 1.0000x reference)
import jax
import jax.numpy as jnp
from jax.experimental import pallas as pl


def kernel(x, shift0, amp0, shift1, amp1):
    raise NotImplementedError("write your pallas kernel here")


# speedup vs baseline: 3.9210x; 3.9210x over previous
import numpy as np
import jax
import jax.numpy as jnp
from jax.experimental import pallas as pl
from jax.experimental.pallas import tpu as pltpu

M = 2049
TAU = 2.821e-5
L = 10.0
FOUR_PI = 4.0 * np.pi
INV_4TAU = 1.0 / (4.0 * TAU)
B, N = 16, 1024
_CHUNK = 128

_j = np.arange(M)
_phase = (np.outer(_j, _j) % M).astype(np.float64) * (2.0 * np.pi / M)
_H64 = (np.cos(_phase) + np.sin(_phase)).astype(np.float32)
_K2 = np.minimum(_j, M - _j).astype(np.float64) ** 2
_D2 = (L / (2.0 * np.pi)) * (np.pi / TAU) * np.exp(2.0 * _K2 * TAU)
_PHI = 2.0 * np.pi * np.linspace(0.0, L, M + 1)[:-1] / L


def _np_split3(x):
    hi = x.astype(np.float32).astype(jnp.bfloat16)
    r = x.astype(np.float32) - np.asarray(hi, np.float32)
    mid = r.astype(jnp.bfloat16)
    lo = (r - np.asarray(mid, np.float32)).astype(jnp.bfloat16)
    return hi, mid, lo


_H_HI_np, _H_MID_np, _H_LO_np = _np_split3(_H64)
_H_HI = jnp.asarray(_H_HI_np)
_H_MID = jnp.asarray(_H_MID_np)
_H_LO = jnp.asarray(_H_LO_np)
_D2_F32 = jnp.asarray(_D2, dtype=jnp.float32)
_K2_F32 = jnp.asarray(_K2, dtype=jnp.float32)
_PHI_ROW = jnp.asarray(_PHI, dtype=jnp.float32).reshape(1, M)


def _bmm(a, b, dims):
    return jax.lax.dot_general(a, b, (dims, ((), ())),
                               preferred_element_type=jnp.float32)


def _split2(x):
    hi = x.astype(jnp.bfloat16)
    lo = (x - hi.astype(jnp.float32)).astype(jnp.bfloat16)
    return hi, lo


def _spread_body(x_ref, phi_ref, gsum_ref):
    phi = phi_ref[0, :][None, :]
    acc = None
    for i in range(N // _CHUNK):
        th = x_ref[0, i * _CHUNK:(i + 1) * _CHUNK] * jnp.float32(2.0 * np.pi / L)
        d = th - phi
        gc = jnp.exp(d * d * jnp.float32(-INV_4TAU))
        p = jnp.sum(gc, axis=0, keepdims=True)
        acc = p if acc is None else acc + p
    gsum_ref[0] = acc


def _spectral_body(gsum_ref, w_ref, hh_ref, hm_ref, hl_ref, irf_ref):
    gsum = gsum_ref[:, 0, :]
    sh = gsum.astype(jnp.bfloat16)
    r = gsum - sh.astype(jnp.float32)
    sm = r.astype(jnp.bfloat16)
    sl = (r - sm.astype(jnp.float32)).astype(jnp.bfloat16)
    hh, hm, hl = hh_ref[:, :], hm_ref[:, :], hl_ref[:, :]
    dims = (((1,), (0,)), ((), ()))
    a = (_bmm(sh, hh, dims[0]) + _bmm(sh, hm, dims[0]) + _bmm(sm, hh, dims[0])
         + _bmm(sh, hl, dims[0]) + _bmm(sl, hh, dims[0]) + _bmm(sm, hm, dims[0]))
    u = (a[:, None, :] * w_ref[:, :][None, :, :]).reshape(2 * B, M)
    uh, ul = _split2(u)
    irf = (_bmm(uh, hh, dims[0]) + _bmm(uh, hm, dims[0])
           + _bmm(ul, hh, dims[0]) + _bmm(ul, hm, dims[0]))
    irf_ref[:, :, :] = (irf * jnp.float32(1.0 / M)).reshape(B, 2, M)


def _interp_body(x_ref, phi_ref, irf_ref, o_ref, gh_ref, gl_ref):
    phi = phi_ref[0, :][None, :]
    for i in range(N // _CHUNK):
        th = x_ref[0, i * _CHUNK:(i + 1) * _CHUNK] * jnp.float32(2.0 * np.pi / L)
        d = th - phi
        gc = jnp.exp(d * d * jnp.float32(-INV_4TAU))
        gch = gc.astype(jnp.bfloat16)
        gh_ref[i * _CHUNK:(i + 1) * _CHUNK, :] = gch
        gl_ref[i * _CHUNK:(i + 1) * _CHUNK, :] = (gc - gch.astype(jnp.float32)).astype(jnp.bfloat16)
    ih, il = _split2(irf_ref[0])
    gh, gl = gh_ref[:, :], gl_ref[:, :]
    dims = ((1,), (1,))
    acc = _bmm(gh, ih, dims) + _bmm(gh, il, dims) + _bmm(gl, ih, dims)
    o_ref[0] = acc * jnp.float32(1.0 / M)


@jax.jit
def kernel(x, shift0, amp0, shift1, amp1):
    m1 = -amp0[0] * FOUR_PI / (_K2_F32 + jnp.square(5.0 * shift0[0]))
    m2 = amp1[0] * FOUR_PI * jnp.square(1.0 / (_K2_F32 + jnp.square(5.0 * shift1[0])))
    w = jnp.stack([m1, m2], axis=0) * _D2_F32
    xt = x.reshape(B, N, 1)

    gsum = pl.pallas_call(
        _spread_body,
        grid=(B,),
        in_specs=[
            pl.BlockSpec((1, N, 1), lambda b: (b, 0, 0)),
            pl.BlockSpec((1, M), lambda b: (0, 0)),
        ],
        out_specs=pl.BlockSpec((1, 1, M), lambda b: (b, 0, 0)),
        out_shape=jax.ShapeDtypeStruct((B, 1, M), jnp.float32),
        compiler_params=pltpu.CompilerParams(
            dimension_semantics=("parallel",),
        ),
    )(xt, _PHI_ROW)

    irf = pl.pallas_call(
        _spectral_body,
        grid=(1,),
        in_specs=[
            pl.BlockSpec((B, 1, M), lambda i: (0, 0, 0)),
            pl.BlockSpec((2, M), lambda i: (0, 0)),
            pl.BlockSpec((M, M), lambda i: (0, 0)),
            pl.BlockSpec((M, M), lambda i: (0, 0)),
            pl.BlockSpec((M, M), lambda i: (0, 0)),
        ],
        out_specs=pl.BlockSpec((B, 2, M), lambda i: (0, 0, 0)),
        out_shape=jax.ShapeDtypeStruct((B, 2, M), jnp.float32),
        compiler_params=pltpu.CompilerParams(
            dimension_semantics=("arbitrary",),
        ),
    )(gsum, w, _H_HI, _H_MID, _H_LO)

    out = pl.pallas_call(
        _interp_body,
        grid=(B,),
        in_specs=[
            pl.BlockSpec((1, N, 1), lambda b: (b, 0, 0)),
            pl.BlockSpec((1, M), lambda b: (0, 0)),
            pl.BlockSpec((1, 2, M), lambda b: (b, 0, 0)),
        ],
        out_specs=pl.BlockSpec((1, N, 2), lambda b: (b, 0, 0)),
        out_shape=jax.ShapeDtypeStruct((B, N, 2), jnp.float32),
        scratch_shapes=[pltpu.VMEM((N, M), jnp.bfloat16),
                        pltpu.VMEM((N, M), jnp.bfloat16)],
        compiler_params=pltpu.CompilerParams(
            dimension_semantics=("parallel",),
        ),
    )(xt, _PHI_ROW, irf)
    return out


# speedup vs baseline: 3.9609x; 1.0102x over previous
import numpy as np
import jax
import jax.numpy as jnp
from jax.experimental import pallas as pl
from jax.experimental.pallas import tpu as pltpu

M = 2049
TAU = 2.821e-5
L = 10.0
FOUR_PI = 4.0 * np.pi
INV_4TAU = 1.0 / (4.0 * TAU)
B, N = 16, 1024
_CHUNK = 128

_j = np.arange(M)
_phase = (np.outer(_j, _j) % M).astype(np.float64) * (2.0 * np.pi / M)
_H64 = np.cos(_phase) + np.sin(_phase)
_K2 = np.minimum(_j, M - _j).astype(np.float64) ** 2
_D2 = (L / (2.0 * np.pi)) * (np.pi / TAU) * np.exp(2.0 * _K2 * TAU)
_PHI = 2.0 * np.pi * np.linspace(0.0, L, M + 1)[:-1] / L

_H_F32 = jnp.asarray(_H64, dtype=jnp.float32)
_H_HI = _H_F32.astype(jnp.bfloat16)
_H_MID = (_H_F32 - _H_HI.astype(jnp.float32)).astype(jnp.bfloat16)
_H_LO = (_H_F32 - _H_HI.astype(jnp.float32)
         - _H_MID.astype(jnp.float32)).astype(jnp.bfloat16)
_D2_F32 = jnp.asarray(_D2, dtype=jnp.float32)
_K2_F32 = jnp.asarray(_K2, dtype=jnp.float32)
_PHI_ROW = jnp.asarray(_PHI, dtype=jnp.float32).reshape(1, M)


def _mm(a, b, dims):
    return jax.lax.dot_general(a, b, (dims, ((), ())),
                               preferred_element_type=jnp.float32)


def _spread_body(x_ref, phi_ref, gsum_ref):
    phi = phi_ref[0, :][None, :]
    acc = None
    for i in range(N // _CHUNK):
        th = x_ref[0, i * _CHUNK:(i + 1) * _CHUNK] * jnp.float32(2.0 * np.pi / L)
        d = th - phi
        gc = jnp.exp(d * d * jnp.float32(-INV_4TAU))
        p = jnp.sum(gc, axis=0, keepdims=True)
        acc = p if acc is None else acc + p
    gsum_ref[0] = acc


def _spectral_body(gsum_ref, w_ref, hh_ref, hm_ref, hl_ref, irf_ref):
    gsum = gsum_ref[:, 0, :]
    sh = gsum.astype(jnp.bfloat16)
    r = gsum - sh.astype(jnp.float32)
    sm = r.astype(jnp.bfloat16)
    sl = (r - sm.astype(jnp.float32)).astype(jnp.bfloat16)
    hh, hm, hl = hh_ref[:, :], hm_ref[:, :], hl_ref[:, :]
    cd = ((1,), (0,))
    a = (_mm(sh, hh, cd) + _mm(sh, hm, cd) + _mm(sm, hh, cd)
         + _mm(sh, hl, cd) + _mm(sl, hh, cd) + _mm(sm, hm, cd))
    u = (a[:, None, :] * w_ref[:, :][None, :, :]).reshape(2 * B, M)
    uh = u.astype(jnp.bfloat16)
    ul = (u - uh.astype(jnp.float32)).astype(jnp.bfloat16)
    irf = _mm(uh, hh, cd) + _mm(uh, hm, cd) + _mm(ul, hh, cd)
    irf_ref[:, :, :] = (irf * jnp.float32(1.0 / M)).reshape(B, 2, M)


def _interp_body(x_ref, phi_ref, irf_ref, o_ref, gh_ref, gl_ref):
    phi = phi_ref[0, :][None, :]
    for i in range(N // _CHUNK):
        th = x_ref[0, i * _CHUNK:(i + 1) * _CHUNK] * jnp.float32(2.0 * np.pi / L)
        d = th - phi
        gc = jnp.exp(d * d * jnp.float32(-INV_4TAU))
        gch = gc.astype(jnp.bfloat16)
        gh_ref[i * _CHUNK:(i + 1) * _CHUNK, :] = gch
        gl_ref[i * _CHUNK:(i + 1) * _CHUNK, :] = (gc - gch.astype(jnp.float32)).astype(jnp.bfloat16)
    ih = irf_ref[0].astype(jnp.bfloat16)
    il = (irf_ref[0] - ih.astype(jnp.float32)).astype(jnp.bfloat16)
    gh, gl = gh_ref[:, :], gl_ref[:, :]
    cd = ((1,), (1,))
    acc = _mm(gh, ih, cd) + _mm(gh, il, cd) + _mm(gl, ih, cd)
    o_ref[0] = acc * jnp.float32(1.0 / M)


@jax.jit
def kernel(x, shift0, amp0, shift1, amp1):
    m1 = -amp0[0] * FOUR_PI / (_K2_F32 + jnp.square(5.0 * shift0[0]))
    m2 = amp1[0] * FOUR_PI * jnp.square(1.0 / (_K2_F32 + jnp.square(5.0 * shift1[0])))
    w = jnp.stack([m1, m2], axis=0) * _D2_F32
    xt = x.reshape(B, N, 1)

    gsum = pl.pallas_call(
        _spread_body,
        grid=(B,),
        in_specs=[
            pl.BlockSpec((1, N, 1), lambda b: (b, 0, 0)),
            pl.BlockSpec((1, M), lambda b: (0, 0)),
        ],
        out_specs=pl.BlockSpec((1, 1, M), lambda b: (b, 0, 0)),
        out_shape=jax.ShapeDtypeStruct((B, 1, M), jnp.float32),
        compiler_params=pltpu.CompilerParams(
            dimension_semantics=("arbitrary",),
        ),
    )(xt, _PHI_ROW)

    irf = pl.pallas_call(
        _spectral_body,
        grid=(1,),
        in_specs=[
            pl.BlockSpec((B, 1, M), lambda i: (0, 0, 0)),
            pl.BlockSpec((2, M), lambda i: (0, 0)),
            pl.BlockSpec((M, M), lambda i: (0, 0)),
            pl.BlockSpec((M, M), lambda i: (0, 0)),
            pl.BlockSpec((M, M), lambda i: (0, 0)),
        ],
        out_specs=pl.BlockSpec((B, 2, M), lambda i: (0, 0, 0)),
        out_shape=jax.ShapeDtypeStruct((B, 2, M), jnp.float32),
        compiler_params=pltpu.CompilerParams(
            dimension_semantics=("arbitrary",),
        ),
    )(gsum, w, _H_HI, _H_MID, _H_LO)

    out = pl.pallas_call(
        _interp_body,
        grid=(B,),
        in_specs=[
            pl.BlockSpec((1, N, 1), lambda b: (b, 0, 0)),
            pl.BlockSpec((1, M), lambda b: (0, 0)),
            pl.BlockSpec((1, 2, M), lambda b: (b, 0, 0)),
        ],
        out_specs=pl.BlockSpec((1, N, 2), lambda b: (b, 0, 0)),
        out_shape=jax.ShapeDtypeStruct((B, N, 2), jnp.float32),
        scratch_shapes=[pltpu.VMEM((N, M), jnp.bfloat16),
                        pltpu.VMEM((N, M), jnp.bfloat16)],
        compiler_params=pltpu.CompilerParams(
            dimension_semantics=("arbitrary",),
        ),
    )(xt, _PHI_ROW, irf)
    return out


# speedup vs baseline: 4.5007x; 1.1363x over previous
import numpy as np
import jax
import jax.numpy as jnp
from jax.experimental import pallas as pl
from jax.experimental.pallas import tpu as pltpu

M = 2049
TAU = 2.821e-5
L = 10.0
FOUR_PI = 4.0 * np.pi
INV_4TAU = 1.0 / (4.0 * TAU)
B, N = 16, 1024
_CHUNK = 128

_j = np.arange(M)
_phase = (np.outer(_j, _j) % M).astype(np.float64) * (2.0 * np.pi / M)
_H64 = np.cos(_phase) + np.sin(_phase)
_K2 = np.minimum(_j, M - _j).astype(np.float64) ** 2
_D2 = (L / (2.0 * np.pi)) * (np.pi / TAU) * np.exp(2.0 * _K2 * TAU)
_PHI = 2.0 * np.pi * np.linspace(0.0, L, M + 1)[:-1] / L

_H_F32 = jnp.asarray(_H64, dtype=jnp.float32)
_H_HI = _H_F32.astype(jnp.bfloat16)
_H_MID = (_H_F32 - _H_HI.astype(jnp.float32)).astype(jnp.bfloat16)
_H_LO = (_H_F32 - _H_HI.astype(jnp.float32)
         - _H_MID.astype(jnp.float32)).astype(jnp.bfloat16)
_D2_F32 = jnp.asarray(_D2, dtype=jnp.float32)
_K2_F32 = jnp.asarray(_K2, dtype=jnp.float32)
_PHI_ROW = jnp.asarray(_PHI, dtype=jnp.float32).reshape(1, M)


def _mm(a, b, dims):
    return jax.lax.dot_general(a, b, (dims, ((), ())),
                               preferred_element_type=jnp.float32)


def _g_chunks(x_ref, phi):
    for i in range(N // _CHUNK):
        th = x_ref[0, i * _CHUNK:(i + 1) * _CHUNK] * jnp.float32(2.0 * np.pi / L)
        d = th - phi
        yield slice(i * _CHUNK, (i + 1) * _CHUNK), jnp.exp(d * d * jnp.float32(-INV_4TAU))


def _body(x_ref, phi_ref, w_ref, hh_ref, hm_ref, hl_ref, o_ref,
          gs_ref, irf_ref, gh_ref, gl_ref):
    s = pl.program_id(0)
    phi = phi_ref[0, :][None, :]

    @pl.when(s < B)
    def _spread():
        acc = None
        for _, gc in _g_chunks(x_ref, phi):
            p = jnp.sum(gc, axis=0, keepdims=True)
            acc = p if acc is None else acc + p
        gs_ref[pl.ds(s, 1)] = acc[None]

    @pl.when(s == B)
    def _spectral():
        gsum = gs_ref[:, 0, :]
        sh = gsum.astype(jnp.bfloat16)
        r = gsum - sh.astype(jnp.float32)
        sm = r.astype(jnp.bfloat16)
        sl = (r - sm.astype(jnp.float32)).astype(jnp.bfloat16)
        hh, hm, hl = hh_ref[:, :], hm_ref[:, :], hl_ref[:, :]
        cd = ((1,), (0,))
        a = (_mm(sh, hh, cd) + _mm(sh, hm, cd) + _mm(sm, hh, cd)
             + _mm(sh, hl, cd) + _mm(sl, hh, cd) + _mm(sm, hm, cd))
        u = (a[:, None, :] * w_ref[:, :][None, :, :]).reshape(2 * B, M)
        uh = u.astype(jnp.bfloat16)
        ul = (u - uh.astype(jnp.float32)).astype(jnp.bfloat16)
        irf = _mm(uh, hh, cd) + _mm(uh, hm, cd) + _mm(ul, hh, cd)
        irf_ref[:, :, :] = (irf * jnp.float32(1.0 / M)).reshape(B, 2, M)

    @pl.when(s > B)
    def _interp():
        b = s - B - 1
        for sl_, gc in _g_chunks(x_ref, phi):
            gch = gc.astype(jnp.bfloat16)
            gh_ref[sl_, :] = gch
            gl_ref[sl_, :] = (gc - gch.astype(jnp.float32)).astype(jnp.bfloat16)
        ir = irf_ref[pl.ds(b, 1)][0]
        ih = ir.astype(jnp.bfloat16)
        il = (ir - ih.astype(jnp.float32)).astype(jnp.bfloat16)
        gh, gl = gh_ref[:, :], gl_ref[:, :]
        cd = ((1,), (1,))
        acc = _mm(gh, ih, cd) + _mm(gh, il, cd) + _mm(gl, ih, cd)
        o_ref[0] = acc * jnp.float32(1.0 / M)


@jax.jit
def kernel(x, shift0, amp0, shift1, amp1):
    m1 = -amp0[0] * FOUR_PI / (_K2_F32 + jnp.square(5.0 * shift0[0]))
    m2 = amp1[0] * FOUR_PI * jnp.square(1.0 / (_K2_F32 + jnp.square(5.0 * shift1[0])))
    w = jnp.stack([m1, m2], axis=0) * _D2_F32
    xt = x.reshape(B, N, 1)

    def _x_idx(s):
        return (jnp.where(s < B, s, jnp.clip(s - B - 1, 0, B - 1)), 0, 0)

    out = pl.pallas_call(
        _body,
        grid=(2 * B + 1,),
        in_specs=[
            pl.BlockSpec((1, N, 1), _x_idx),
            pl.BlockSpec((1, M), lambda s: (0, 0)),
            pl.BlockSpec((2, M), lambda s: (0, 0)),
            pl.BlockSpec((M, M), lambda s: (0, 0)),
            pl.BlockSpec((M, M), lambda s: (0, 0)),
            pl.BlockSpec((M, M), lambda s: (0, 0)),
        ],
        out_specs=pl.BlockSpec((1, N, 2),
                               lambda s: (jnp.clip(s - B - 1, 0, B - 1), 0, 0)),
        out_shape=jax.ShapeDtypeStruct((B, N, 2), jnp.float32),
        scratch_shapes=[
            pltpu.VMEM((B, 1, M), jnp.float32),
            pltpu.VMEM((B, 2, M), jnp.float32),
            pltpu.VMEM((N, M), jnp.bfloat16),
            pltpu.VMEM((N, M), jnp.bfloat16),
        ],
        compiler_params=pltpu.CompilerParams(
            dimension_semantics=("arbitrary",),
        ),
    )(xt, _PHI_ROW, w, _H_HI, _H_MID, _H_LO)
    return out


# speedup vs baseline: 4.8247x; 1.0720x over previous
import numpy as np
import jax
import jax.numpy as jnp
from jax.experimental import pallas as pl
from jax.experimental.pallas import tpu as pltpu

M = 2049
TAU = 2.821e-5
L = 10.0
FOUR_PI = 4.0 * np.pi
INV_4TAU = 1.0 / (4.0 * TAU)
B, N = 16, 1024
_CHUNK = 128

_j = np.arange(M)
_phase = (np.outer(_j, _j) % M).astype(np.float64) * (2.0 * np.pi / M)
_H64 = np.cos(_phase) + np.sin(_phase)
_K2 = np.minimum(_j, M - _j).astype(np.float64) ** 2
_D2 = (L / (2.0 * np.pi)) * (np.pi / TAU) * np.exp(2.0 * _K2 * TAU)
_PHI = 2.0 * np.pi * np.linspace(0.0, L, M + 1)[:-1] / L
_SQ = np.sqrt(np.log2(np.e) / (4.0 * TAU))
_XSCALE = 2.0 * np.pi / L * _SQ

_H_F32 = jnp.asarray(_H64, dtype=jnp.float32)
_H_HI = _H_F32.astype(jnp.bfloat16)
_H_MID = (_H_F32 - _H_HI.astype(jnp.float32)).astype(jnp.bfloat16)
_H_LO = (_H_F32 - _H_HI.astype(jnp.float32)
         - _H_MID.astype(jnp.float32)).astype(jnp.bfloat16)
_D2_F32 = jnp.asarray(_D2, dtype=jnp.float32)
_K2_F32 = jnp.asarray(_K2, dtype=jnp.float32)
_PHI_ROW = jnp.asarray(_PHI * _SQ, dtype=jnp.float32).reshape(1, M)


def _mm(a, b, dims):
    return jax.lax.dot_general(a, b, (dims, ((), ())),
                               preferred_element_type=jnp.float32)


def _g_chunks(x_ref, phi):
    for i in range(N // _CHUNK):
        th = x_ref[0, i * _CHUNK:(i + 1) * _CHUNK] * jnp.float32(_XSCALE)
        a = th - phi
        yield slice(i * _CHUNK, (i + 1) * _CHUNK), jnp.exp2(a * (phi - th))


def _body(x_ref, phi_ref, w_ref, hh_ref, hm_ref, hl_ref, o_ref,
          gs_ref, irf_ref, gh_ref, gl_ref, hhs_ref, hms_ref, hls_ref, sem):
    s = pl.program_id(0)
    phi = phi_ref[0, :][None, :]

    @pl.when(s == 0)
    def _h_fetch():
        pltpu.make_async_copy(hh_ref, hhs_ref, sem.at[0]).start()
        pltpu.make_async_copy(hm_ref, hms_ref, sem.at[1]).start()
        pltpu.make_async_copy(hl_ref, hls_ref, sem.at[2]).start()

    @pl.when(s < B)
    def _spread():
        acc = None
        for _, gc in _g_chunks(x_ref, phi):
            p = jnp.sum(gc, axis=0, keepdims=True)
            acc = p if acc is None else acc + p
        gs_ref[pl.ds(s, 1)] = acc[None]

    @pl.when(s == B)
    def _spectral():
        gsum = gs_ref[:, 0, :]
        sh = gsum.astype(jnp.bfloat16)
        r = gsum - sh.astype(jnp.float32)
        sm = r.astype(jnp.bfloat16)
        sl = (r - sm.astype(jnp.float32)).astype(jnp.bfloat16)
        pltpu.make_async_copy(hh_ref, hhs_ref, sem.at[0]).wait()
        pltpu.make_async_copy(hm_ref, hms_ref, sem.at[1]).wait()
        pltpu.make_async_copy(hl_ref, hls_ref, sem.at[2]).wait()
        hh, hm, hl = hhs_ref[:, :], hms_ref[:, :], hls_ref[:, :]
        cd = ((1,), (0,))
        a = (_mm(sh, hh, cd) + _mm(sh, hm, cd) + _mm(sm, hh, cd)
             + _mm(sh, hl, cd) + _mm(sl, hh, cd) + _mm(sm, hm, cd))
        u = (a[:, None, :] * w_ref[:, :][None, :, :]).reshape(2 * B, M)
        uh = u.astype(jnp.bfloat16)
        ul = (u - uh.astype(jnp.float32)).astype(jnp.bfloat16)
        irf = _mm(uh, hh, cd) + _mm(uh, hm, cd) + _mm(ul, hh, cd)
        irf_ref[:, :, :] = (irf * jnp.float32(1.0 / M)).reshape(B, 2, M)

    @pl.when(s > B)
    def _interp():
        b = s - B - 1
        for sl_, gc in _g_chunks(x_ref, phi):
            gch = gc.astype(jnp.bfloat16)
            gh_ref[sl_, :] = gch
            gl_ref[sl_, :] = (gc - gch.astype(jnp.float32)).astype(jnp.bfloat16)
        ir = irf_ref[pl.ds(b, 1)][0]
        ih = ir.astype(jnp.bfloat16)
        il = (ir - ih.astype(jnp.float32)).astype(jnp.bfloat16)
        gh, gl = gh_ref[:, :], gl_ref[:, :]
        cd = ((1,), (1,))
        acc = _mm(gh, ih, cd) + _mm(gh, il, cd) + _mm(gl, ih, cd)
        o_ref[0] = acc * jnp.float32(1.0 / M)


@jax.jit
def kernel(x, shift0, amp0, shift1, amp1):
    m1 = -amp0[0] * FOUR_PI / (_K2_F32 + jnp.square(5.0 * shift0[0]))
    m2 = amp1[0] * FOUR_PI * jnp.square(1.0 / (_K2_F32 + jnp.square(5.0 * shift1[0])))
    w = jnp.stack([m1, m2], axis=0) * _D2_F32
    xt = x.reshape(B, N, 1)

    def _x_idx(s):
        return (jnp.where(s < B, s, jnp.clip(s - B - 1, 0, B - 1)), 0, 0)

    out = pl.pallas_call(
        _body,
        grid=(2 * B + 1,),
        in_specs=[
            pl.BlockSpec((1, N, 1), _x_idx),
            pl.BlockSpec((1, M), lambda s: (0, 0)),
            pl.BlockSpec((2, M), lambda s: (0, 0)),
            pl.BlockSpec(memory_space=pl.ANY),
            pl.BlockSpec(memory_space=pl.ANY),
            pl.BlockSpec(memory_space=pl.ANY),
        ],
        out_specs=pl.BlockSpec((1, N, 2),
                               lambda s: (jnp.clip(s - B - 1, 0, B - 1), 0, 0)),
        out_shape=jax.ShapeDtypeStruct((B, N, 2), jnp.float32),
        scratch_shapes=[
            pltpu.VMEM((B, 1, M), jnp.float32),
            pltpu.VMEM((B, 2, M), jnp.float32),
            pltpu.VMEM((N, M), jnp.bfloat16),
            pltpu.VMEM((N, M), jnp.bfloat16),
            pltpu.VMEM((M, M), jnp.bfloat16),
            pltpu.VMEM((M, M), jnp.bfloat16),
            pltpu.VMEM((M, M), jnp.bfloat16),
            pltpu.SemaphoreType.DMA((3,)),
        ],
        compiler_params=pltpu.CompilerParams(
            dimension_semantics=("arbitrary",),
        ),
    )(xt, _PHI_ROW, w, _H_HI, _H_MID, _H_LO)
    return out
